# Initial kernel scaffold; baseline (speedup 1.0000x reference)
import jax
import jax.numpy as jnp
from jax.experimental import pallas as pl


def kernel(x, edge_index, batch_index, Wl0, Wr0, a0, b0, Wls, Wrs, atts, bs, Wout, bout):
    raise NotImplementedError("write your pallas kernel here")


# speedup vs baseline: 2.0915x; 2.0915x over previous
import functools

import jax
import jax.numpy as jnp
from jax import lax
from jax.experimental import pallas as pl
from jax.experimental.pallas import tpu as pltpu
from jax.experimental.pallas import tpu_sc as plsc

N = 10000
FEAT = 512
NG = 64
NCLS = 10
NC = 2
NS = 16
NW = NC * NS
LANE = 16
CH = 64
TILE_ROWS = 320
NCHUNK = TILE_ROWS // CH
NEG_INF = float("-inf")


def _seg_max_body(x_hbm, ids_hbm, part_hbm, xbuf, ids_v, acc):
    wid = lax.axis_index("c") * NS + lax.axis_index("s")

    def init_g(g, carry):
        for c in range(FEAT // LANE):
            acc[g, pl.ds(c * LANE, LANE)] = jnp.full((LANE,), NEG_INF, jnp.float32)
        return carry

    lax.fori_loop(0, NG, init_g, 0)

    base0 = wid * TILE_ROWS

    def chunk_body(k, carry):
        base = jnp.minimum(base0 + k * CH, N - CH)
        pltpu.sync_copy(x_hbm.at[pl.ds(base, CH)], xbuf)
        pltpu.sync_copy(ids_hbm.at[pl.ds(base, CH)], ids_v)

        def group_body(rb, carry):
            idvec = ids_v[pl.ds(rb * LANE, LANE)]
            for j in range(LANE):
                g = idvec[j]
                r = rb * LANE + j
                for c in range(FEAT // LANE):
                    sl = pl.ds(c * LANE, LANE)
                    acc[g, sl] = jnp.maximum(acc[g, sl], xbuf[r, sl])
            return carry

        lax.fori_loop(0, CH // LANE, group_body, 0)
        return carry

    lax.fori_loop(0, NCHUNK, chunk_body, 0)
    pltpu.sync_copy(acc, part_hbm.at[wid])


@functools.cache
def _seg_max():
    return functools.partial(
        pl.kernel,
        out_type=jax.ShapeDtypeStruct((NW, NG, FEAT), jnp.float32),
        mesh=plsc.VectorSubcoreMesh(
            core_axis_name="c", subcore_axis_name="s",
            num_cores=NC, num_subcores=NS,
        ),
        scratch_types=[
            pltpu.VMEM((CH, FEAT), jnp.float32),
            pltpu.VMEM((CH,), jnp.int32),
            pltpu.VMEM((NG, FEAT), jnp.float32),
        ],
    )(_seg_max_body)


def _finish_body(part_ref, w_ref, b_ref, out_ref, pooled_ref):
    p = part_ref[0]
    for i in range(1, NW):
        p = jnp.maximum(p, part_ref[i])
    pooled_ref[...] = p
    out_ref[...] = (
        jnp.dot(p, w_ref[...], preferred_element_type=jnp.float32) + b_ref[...]
    )


def kernel(x, edge_index, batch_index, Wl0, Wr0, a0, b0, Wls, Wrs, atts, bs,
           Wout, bout):
    partials = _seg_max()(x, batch_index)
    out, pooled = pl.pallas_call(
        _finish_body,
        out_shape=(
            jax.ShapeDtypeStruct((NG, NCLS), jnp.float32),
            jax.ShapeDtypeStruct((NG, FEAT), jnp.float32),
        ),
    )(partials, Wout, bout.reshape(1, NCLS))
    return (out, pooled)


# speedup vs baseline: 3.8941x; 1.8619x over previous
import functools

import jax
import jax.numpy as jnp
from jax import lax
from jax.experimental import pallas as pl
from jax.experimental.pallas import tpu as pltpu
from jax.experimental.pallas import tpu_sc as plsc

N = 10000
FEAT = 512
NG = 64
NCLS = 10
NC = 2
NS = 16
NW = NC * NS
LANE = 16
CH = 64
TILE_ROWS = 320
NCHUNK = TILE_ROWS // CH
NEG_INF = float("-inf")


NCHW = FEAT // LANE


def _flush(acc, g_cur, regs):
    for c in range(NCHW):
        sl = pl.ds(c * LANE, LANE)
        acc[g_cur, sl] = jnp.maximum(acc[g_cur, sl], regs[c])


def _seg_max_body(x_hbm, ids_hbm, part_hbm, xbuf, ids_v, acc):
    wid = lax.axis_index("c") * NS + lax.axis_index("s")

    def init_g(g, carry):
        for c in range(NCHW):
            acc[g, pl.ds(c * LANE, LANE)] = jnp.full((LANE,), NEG_INF, jnp.float32)
        return carry

    lax.fori_loop(0, NG, init_g, 0)

    base0 = wid * TILE_ROWS
    neg = jnp.full((LANE,), NEG_INF, jnp.float32)
    carry0 = (jnp.int32(0),) + (neg,) * NCHW

    def chunk_body(k, carry):
        base = jnp.minimum(base0 + k * CH, N - CH)
        pltpu.sync_copy(x_hbm.at[pl.ds(base, CH)], xbuf)
        pltpu.sync_copy(ids_hbm.at[pl.ds(base, CH)], ids_v)

        def group_body(rb, carry):
            g_cur = carry[0]
            regs = list(carry[1:])
            idvec = ids_v[pl.ds(rb * LANE, LANE)]
            for j in range(LANE):
                g = idvec[j]
                changed = g != g_cur

                @pl.when(changed)
                def _():
                    _flush(acc, g_cur, regs)

                r = rb * LANE + j
                for c in range(NCHW):
                    row_c = xbuf[r, pl.ds(c * LANE, LANE)]
                    regs[c] = jnp.where(
                        changed, row_c, jnp.maximum(regs[c], row_c)
                    )
                g_cur = g
            return (g_cur,) + tuple(regs)

        return lax.fori_loop(0, CH // LANE, group_body, carry)

    carry = lax.fori_loop(0, NCHUNK, chunk_body, carry0)
    _flush(acc, carry[0], list(carry[1:]))
    pltpu.sync_copy(acc, part_hbm.at[wid])


@functools.cache
def _seg_max():
    return functools.partial(
        pl.kernel,
        out_type=jax.ShapeDtypeStruct((NW, NG, FEAT), jnp.float32),
        mesh=plsc.VectorSubcoreMesh(
            core_axis_name="c", subcore_axis_name="s",
            num_cores=NC, num_subcores=NS,
        ),
        scratch_types=[
            pltpu.VMEM((CH, FEAT), jnp.float32),
            pltpu.VMEM((CH,), jnp.int32),
            pltpu.VMEM((NG, FEAT), jnp.float32),
        ],
    )(_seg_max_body)


def _finish_body(part_ref, w_ref, b_ref, out_ref, pooled_ref):
    p = part_ref[0]
    for i in range(1, NW):
        p = jnp.maximum(p, part_ref[i])
    pooled_ref[...] = p
    out_ref[...] = (
        jnp.dot(p, w_ref[...], preferred_element_type=jnp.float32) + b_ref[...]
    )


def kernel(x, edge_index, batch_index, Wl0, Wr0, a0, b0, Wls, Wrs, atts, bs,
           Wout, bout):
    partials = _seg_max()(x, batch_index)
    out, pooled = pl.pallas_call(
        _finish_body,
        out_shape=(
            jax.ShapeDtypeStruct((NG, NCLS), jnp.float32),
            jax.ShapeDtypeStruct((NG, FEAT), jnp.float32),
        ),
    )(partials, Wout, bout.reshape(1, NCLS))
    return (out, pooled)
